# baseline (speedup 1.0000x reference)
import functools

import jax
import jax.numpy as jnp
from jax import lax
from jax.experimental import pallas as pl
from jax.experimental.pallas import tpu as pltpu
from jax.experimental.pallas import tpu_sc as plsc

_B = 1024
_D = 16
_NBITS = 32


def _spike_embed_call(token_ids, weight_float):
    info = plsc.get_sparse_core_info()
    nc, ns, nl = info.num_cores, info.num_subcores, info.num_lanes
    nw = nc * ns
    bpw = _B // nw

    mesh = plsc.VectorSubcoreMesh(core_axis_name="c", subcore_axis_name="s")

    @functools.partial(
        pl.kernel,
        mesh=mesh,
        out_type=jax.ShapeDtypeStruct((_B, _D, _NBITS), jnp.float32),
        scratch_types=[
            pltpu.VMEM((bpw,), jnp.int32),
            pltpu.VMEM((bpw, _D), jnp.float32),
            pltpu.VMEM((bpw, _D, _NBITS), jnp.float32),
            pltpu.SemaphoreType.DMA,
        ],
        compiler_params=pltpu.CompilerParams(
            needs_layout_passes=False, use_tc_tiling_on_sc=False,
            skip_device_barrier=True),
    )
    def spike_embed(ids_hbm, table_hbm, out_hbm, idx_v, rows_v, out_v, sem):
        wid = lax.axis_index("s") * nc + lax.axis_index("c")
        base = wid * bpw
        pltpu.sync_copy(ids_hbm.at[pl.ds(base, bpw)], idx_v)
        pltpu.async_copy(table_hbm.at[idx_v], rows_v, sem).wait()

        lane = lax.iota(jnp.int32, nl)
        shifts = [31 - lane, 15 - lane]
        zeros = jnp.zeros((nl,), jnp.int32)

        def token_body(t, carry):
            row = plsc.bitcast(rows_v[t], jnp.int32)
            for d in range(_D):
                word = zeros + row[d]
                for h in range(2):
                    bits = ((word >> shifts[h]) & 1).astype(jnp.float32)
                    out_v[t, d, pl.ds(h * nl, nl)] = bits
            return carry

        lax.fori_loop(0, bpw, token_body, 0)
        pltpu.sync_copy(out_v, out_hbm.at[pl.ds(base, bpw)])

    return spike_embed(token_ids, weight_float)


def kernel(token_ids, weight_float):
    return _spike_embed_call(token_ids.astype(jnp.int32),
                             weight_float.astype(jnp.float32))


# speedup vs baseline: 1.3634x; 1.3634x over previous
import functools

import jax
import jax.numpy as jnp
from jax import lax
from jax.experimental import pallas as pl
from jax.experimental.pallas import tpu as pltpu
from jax.experimental.pallas import tpu_sc as plsc

_B = 1024
_D = 16
_NBITS = 32


def _spike_embed_call(token_ids, weight_float):
    info = plsc.get_sparse_core_info()
    nc, ns, nl = info.num_cores, info.num_subcores, info.num_lanes
    nw = nc * ns
    bpw = _B // nw

    mesh = plsc.VectorSubcoreMesh(core_axis_name="c", subcore_axis_name="s")

    @functools.partial(
        pl.kernel,
        mesh=mesh,
        out_type=jax.ShapeDtypeStruct((_B, _D * _NBITS), jnp.float32),
        scratch_types=[
            pltpu.VMEM((bpw,), jnp.int32),
            pltpu.VMEM((bpw, _D), jnp.float32),
            pltpu.VMEM((bpw, _D * _NBITS), jnp.float32),
            pltpu.SemaphoreType.DMA,
        ],
        compiler_params=pltpu.CompilerParams(
            needs_layout_passes=False, use_tc_tiling_on_sc=False,
            skip_device_barrier=True),
    )
    def spike_embed(ids_hbm, table_hbm, out_hbm, idx_v, rows_v, out_v, sem):
        wid = lax.axis_index("s") * nc + lax.axis_index("c")
        base = wid * bpw
        pltpu.sync_copy(ids_hbm.at[pl.ds(base, bpw)], idx_v)
        pltpu.async_copy(table_hbm.at[idx_v], rows_v, sem).wait()

        lane = lax.iota(jnp.int32, nl)
        shifts = [31 - lane, 15 - lane]
        zeros = jnp.zeros((nl,), jnp.int32)

        def token_body(t, carry):
            row = plsc.bitcast(rows_v[t], jnp.int32)
            for d in range(_D):
                word = zeros + row[d]
                for h in range(2):
                    bits = ((word >> shifts[h]) & 1).astype(jnp.float32)
                    out_v[t, pl.ds(d * _NBITS + h * nl, nl)] = bits
            return carry

        lax.fori_loop(0, bpw, token_body, 0)
        pltpu.sync_copy(out_v, out_hbm.at[pl.ds(base, bpw)])

    return spike_embed(token_ids, weight_float)


def kernel(token_ids, weight_float):
    out = _spike_embed_call(token_ids.astype(jnp.int32),
                            weight_float.astype(jnp.float32))
    return out.reshape(_B, _D, _NBITS)
